# baseline (speedup 1.0000x reference)
import functools

import jax
import jax.numpy as jnp
from jax import lax
from jax.experimental import pallas as pl
from jax.experimental.pallas import tpu as pltpu
from jax.experimental.pallas import tpu_sc as plsc

_HI = jax.lax.Precision.HIGHEST

NC = 2
NS = 16
LN = 16
FH = 128
CH = 64
SB = 16
CPT = 320
EPT = CPT * CH
EPAD = NS * EPT
NPR = 632
NPAD = NS * NPR


def _mlp_body(x_ref, w1_ref, b1_ref, w2_ref, b2_ref, o_ref):
    h = jnp.maximum(
        jnp.dot(x_ref[...], w1_ref[...], preferred_element_type=jnp.float32, precision=_HI)
        + b1_ref[...], 0.0)
    o_ref[...] = (
        jnp.dot(h, w2_ref[...], preferred_element_type=jnp.float32, precision=_HI) + b2_ref[...])


def _mlp(x, w1, b1, w2, b2, rb):
    r, din = x.shape
    dh = w1.shape[1]
    dout = w2.shape[1]
    return pl.pallas_call(
        _mlp_body,
        grid=(r // rb,),
        in_specs=[
            pl.BlockSpec((rb, din), lambda i: (i, 0)),
            pl.BlockSpec((din, dh), lambda i: (0, 0)),
            pl.BlockSpec((1, dh), lambda i: (0, 0)),
            pl.BlockSpec((dh, dout), lambda i: (0, 0)),
            pl.BlockSpec((1, dout), lambda i: (0, 0)),
        ],
        out_specs=pl.BlockSpec((rb, dout), lambda i: (i, 0)),
        out_shape=jax.ShapeDtypeStruct((r, dout), jnp.float32),
    )(x, w1, b1, w2, b2)


def _mlp2half_body(x_ref, w1_ref, b1_ref, w2_ref, b2_ref, o_ref):
    h = jnp.maximum(
        jnp.dot(x_ref[...], w1_ref[...], preferred_element_type=jnp.float32, precision=_HI)
        + b1_ref[...], 0.0)
    res = jnp.dot(h, w2_ref[...], preferred_element_type=jnp.float32, precision=_HI) + b2_ref[...]
    o_ref[0] = res[:, :FH]
    o_ref[1] = res[:, FH:]


def _mlp2half(x, w1, b1, w2, b2, rb):
    r, din = x.shape
    dh = w1.shape[1]
    return pl.pallas_call(
        _mlp2half_body,
        grid=(r // rb,),
        in_specs=[
            pl.BlockSpec((rb, din), lambda i: (i, 0)),
            pl.BlockSpec((din, dh), lambda i: (0, 0)),
            pl.BlockSpec((1, dh), lambda i: (0, 0)),
            pl.BlockSpec((dh, 2 * FH), lambda i: (0, 0)),
            pl.BlockSpec((1, 2 * FH), lambda i: (0, 0)),
        ],
        out_specs=pl.BlockSpec((2, rb, FH), lambda i: (0, i, 0)),
        out_shape=jax.ShapeDtypeStruct((2, r, FH), jnp.float32),
    )(x, w1, b1, w2, b2)


def _ab_body(x_ref, w_ref, o_ref):
    ab = jnp.dot(x_ref[...], w_ref[...], preferred_element_type=jnp.float32, precision=_HI)
    for k in range(4):
        o_ref[k] = ab[:, k * FH:(k + 1) * FH]


def _ab_tables(v, w_ab, rb):
    r = v.shape[0]
    return pl.pallas_call(
        _ab_body,
        grid=(r // rb,),
        in_specs=[
            pl.BlockSpec((rb, v.shape[1]), lambda i: (i, 0)),
            pl.BlockSpec((v.shape[1], 4 * FH), lambda i: (0, 0)),
        ],
        out_specs=pl.BlockSpec((4, rb, FH), lambda i: (0, i, 0)),
        out_shape=jax.ShapeDtypeStruct((4, r, FH), jnp.float32),
    )(v, w_ab)


def _wprep_body(w2e_ref, b2e_ref, w1e_ref, b1_ref, w2_ref, b2_ref, w1vb_ref,
                wc_ref, cc_ref, wg_ref, u_ref):
    wc_ref[...] = jnp.dot(w2e_ref[...], w1e_ref[...],
                          preferred_element_type=jnp.float32, precision=_HI)
    cc_ref[...] = jnp.dot(b2e_ref[...], w1e_ref[...],
                          preferred_element_type=jnp.float32, precision=_HI) + b1_ref[...]
    wg_ref[...] = jnp.dot(w2_ref[...], w1vb_ref[...],
                          preferred_element_type=jnp.float32, precision=_HI)
    u_ref[...] = jnp.dot(b2_ref[...], w1vb_ref[...],
                         preferred_element_type=jnp.float32, precision=_HI)


def _wprep(w2e, b2e, w1e, b1, w2, b2, w1vb):
    d = w2e.shape[0]
    return pl.pallas_call(
        _wprep_body,
        out_shape=[
            jax.ShapeDtypeStruct((d, d), jnp.float32),
            jax.ShapeDtypeStruct((1, d), jnp.float32),
            jax.ShapeDtypeStruct((d, d), jnp.float32),
            jax.ShapeDtypeStruct((1, d), jnp.float32),
        ],
    )(w2e, b2e, w1e, b1, w2, b2, w1vb)


def _nodeupd_body(v_ref, h0_ref, h1_ref, dg_ref, w1va_ref, wg_ref, u_ref,
                  b1v_ref, w2v_ref, b2v_ref, o_ref):
    wg = wg_ref[...]
    z = (jnp.dot(v_ref[...], w1va_ref[...], preferred_element_type=jnp.float32, precision=_HI)
         + jnp.dot(h0_ref[...], wg[:FH, :], preferred_element_type=jnp.float32, precision=_HI)
         + jnp.dot(h1_ref[...], wg[FH:, :], preferred_element_type=jnp.float32, precision=_HI)
         + dg_ref[:, 0:1] * u_ref[...]
         + b1v_ref[...])
    o_ref[...] = (jnp.dot(jnp.maximum(z, 0.0), w2v_ref[...],
                          preferred_element_type=jnp.float32, precision=_HI) + b2v_ref[...])


def _nodeupd(v, h0, h1, dg, w1va, wg, u, b1v, w2v, b2v, rb):
    r, d = v.shape
    return pl.pallas_call(
        _nodeupd_body,
        grid=(r // rb,),
        in_specs=[
            pl.BlockSpec((rb, d), lambda i: (i, 0)),
            pl.BlockSpec((rb, FH), lambda i: (i, 0)),
            pl.BlockSpec((rb, FH), lambda i: (i, 0)),
            pl.BlockSpec((rb, LN), lambda i: (i, 0)),
            pl.BlockSpec((d, d), lambda i: (0, 0)),
            pl.BlockSpec((d, d), lambda i: (0, 0)),
            pl.BlockSpec((1, d), lambda i: (0, 0)),
            pl.BlockSpec((1, d), lambda i: (0, 0)),
            pl.BlockSpec((d, d), lambda i: (0, 0)),
            pl.BlockSpec((1, d), lambda i: (0, 0)),
        ],
        out_specs=pl.BlockSpec((rb, d), lambda i: (i, 0)),
        out_shape=jax.ShapeDtypeStruct((r, d), jnp.float32),
    )(v, h0, h1, dg, w1va, wg, u, b1v, w2v, b2v)


def _edge_sc(ab, c2, src3d, dst3d):
    ng = CPT // SB
    mesh = plsc.VectorSubcoreMesh(core_axis_name="c", subcore_axis_name="s")

    def body(ab_hbm, c_hbm, src_hbm, dst_hbm, hs_out,
             idx_s, idx_d, a0, a1, b0, b1, cbuf, sa0, sa1, sb0, sb1, acc):
        c = lax.axis_index("c")
        s = lax.axis_index("s")
        abuf = [a0, a1]
        bbuf = [b0, b1]
        sa = [sa0, sa1]
        sb = [sb0, sb1]

        @plsc.parallel_loop(0, 8)
        def _fz(i):
            for k in range(FH // LN):
                cbuf[i, pl.ds(k * LN, LN)] = jnp.zeros((LN,), jnp.float32)

        def _zero(m, carry):
            pltpu.sync_copy(cbuf.at[pl.ds(0, 8)],
                            acc.at[pl.ds(s * NPR + m * 8, 8)])
            return carry
        lax.fori_loop(0, NPR // 8, _zero, 0)
        plsc.subcore_barrier()

        def issue_gather(p, sidx, didx):
            pltpu.async_copy(ab_hbm.at[c].at[sidx], abuf[p], sa[p])
            pltpu.async_copy(ab_hbm.at[c + 2].at[didx], bbuf[p], sb[p])

        def wait_gather(p):
            pltpu.make_async_copy(ab_hbm.at[c].at[idx_s.at[0]],
                                  abuf[p], sa[p]).wait()
            pltpu.make_async_copy(ab_hbm.at[c].at[idx_d.at[0]],
                                  bbuf[p], sb[p]).wait()

        pltpu.sync_copy(src_hbm.at[s, pl.ds(0, SB)], idx_s)
        pltpu.sync_copy(dst_hbm.at[s, pl.ds(0, SB)], idx_d)
        issue_gather(0, idx_s.at[0], idx_d.at[0])

        def sb_body(g, carry):
            for j2 in range(SB):
                p = j2 & 1
                if j2 < SB - 1:
                    issue_gather(1 - p, idx_s.at[j2 + 1], idx_d.at[j2 + 1])
                pltpu.sync_copy(
                    c_hbm.at[c, pl.ds(s * EPT + (g * SB + j2) * CH, CH)], cbuf)
                wait_gather(p)
                va = abuf[p]
                vb = bbuf[p]

                @plsc.parallel_loop(0, CH, unroll=2)
                def _row(r2):
                    for k in range(FH // LN):
                        sl = pl.ds(k * LN, LN)
                        va[r2, sl] = jnp.maximum(
                            va[r2, sl] + vb[r2, sl] + cbuf[r2, sl], 0.0)

                pltpu.sync_copy(va, acc.at[idx_d.at[j2]], add=True)

            @pl.when(g < ng - 1)
            def _():
                pltpu.sync_copy(src_hbm.at[s, pl.ds((g + 1) * SB, SB)], idx_s)
                pltpu.sync_copy(dst_hbm.at[s, pl.ds((g + 1) * SB, SB)], idx_d)
                issue_gather(0, idx_s.at[0], idx_d.at[0])
            return carry
        lax.fori_loop(0, ng, sb_body, 0)
        plsc.subcore_barrier()

        pltpu.sync_copy(acc.at[pl.ds(s * NPR, NPR)],
                        hs_out.at[c, pl.ds(s * NPR, NPR)])

    call = pl.kernel(
        body,
        out_type=jax.ShapeDtypeStruct((NC, NPAD, FH), jnp.float32),
        mesh=mesh,
        scratch_types=[
            pltpu.VMEM((SB, CH), jnp.int32),
            pltpu.VMEM((SB, CH), jnp.int32),
            pltpu.VMEM((CH, FH), jnp.float32),
            pltpu.VMEM((CH, FH), jnp.float32),
            pltpu.VMEM((CH, FH), jnp.float32),
            pltpu.VMEM((CH, FH), jnp.float32),
            pltpu.VMEM((CH, FH), jnp.float32),
            pltpu.SemaphoreType.DMA,
            pltpu.SemaphoreType.DMA,
            pltpu.SemaphoreType.DMA,
            pltpu.SemaphoreType.DMA,
            pltpu.VMEM_SHARED((NPAD, FH), jnp.float32),
        ],
    )
    return call(ab, c2, src3d, dst3d)


def _deg_sc(dst3d):
    mesh = plsc.VectorSubcoreMesh(core_axis_name="c", subcore_axis_name="s")

    def body(dst_hbm, deg_out, idx_d, ones_b, zrow16, dacc):
        c = lax.axis_index("c")
        s = lax.axis_index("s")

        def _fill(i, carry):
            ones_b[i, :] = jnp.ones((LN,), jnp.float32)
            zrow16[i % 8, :] = jnp.zeros((LN,), jnp.float32)
            return carry
        lax.fori_loop(0, CH, _fill, 0)

        def _zero(m, carry):
            pltpu.sync_copy(zrow16, dacc.at[pl.ds(s * NPR + m * 8, 8)])
            return carry
        lax.fori_loop(0, NPR // 8, _zero, 0)
        plsc.subcore_barrier()

        def superblock(g, carry):
            pltpu.sync_copy(dst_hbm.at[s, pl.ds(g * SB, SB)], idx_d)

            def chunk(j2, carry2):
                pltpu.sync_copy(ones_b, dacc.at[idx_d.at[j2]], add=True)
                return carry2
            lax.fori_loop(0, SB, chunk, 0)
            return carry
        lax.fori_loop(0, CPT // SB, superblock, 0)
        plsc.subcore_barrier()

        @pl.when(c == 0)
        def _():
            pltpu.sync_copy(dacc.at[pl.ds(s * NPR, NPR)],
                            deg_out.at[pl.ds(s * NPR, NPR)])

    call = pl.kernel(
        body,
        out_type=jax.ShapeDtypeStruct((NPAD, LN), jnp.float32),
        mesh=mesh,
        scratch_types=[
            pltpu.VMEM((SB, CH), jnp.int32),
            pltpu.VMEM((CH, LN), jnp.float32),
            pltpu.VMEM((8, LN), jnp.float32),
            pltpu.VMEM_SHARED((NPAD, LN), jnp.float32),
        ],
    )
    return call(dst3d)


def kernel(vertexes, edge_index, edge_attr, params):
    n = vertexes.shape[0]
    e = edge_attr.shape[0]
    dh = params["enc_v"]["w2"].shape[1]

    def b2d(b):
        return b.reshape(1, -1)

    pad_e = EPAD - e
    idx_pad = jnp.full((2, pad_e), NPAD - 1, jnp.int32)
    eidx = jnp.concatenate([edge_index, idx_pad], axis=1)
    src3d = eidx[0].reshape(NS, CPT, CH)
    dst3d = eidx[1].reshape(NS, CPT, CH)
    ea_pad = jnp.pad(edge_attr, ((0, pad_e), (0, 0)))

    pv = params["enc_v"]
    v = _mlp(vertexes, pv["w1"], b2d(pv["b1"]), pv["w2"], b2d(pv["b2"]), rb=1000)

    pe = params["enc_e"]
    w1e_enc, b1e_enc = pe["w1"], b2d(pe["b1"])
    w2e_enc, b2e_enc = pe["w2"], b2d(pe["b2"])

    deg = _deg_sc(dst3d)[:n]

    for p in params["mpb"]:
        w1_pe = p["psi_e"]["w1"]
        w1s = w1_pe[:dh]
        w1d = w1_pe[dh:2 * dh]
        w1e = w1_pe[2 * dh:]
        w1_pv = p["psi_v"]["w1"]
        w1va = w1_pv[:dh]
        w1vb = w1_pv[dh:]

        wc, cc, wg, u = _wprep(
            w2e_enc, b2e_enc, w1e, b2d(p["psi_e"]["b1"]),
            p["psi_e"]["w2"], b2d(p["psi_e"]["b2"]), w1vb)

        w_ab = jnp.concatenate([w1s, w1d], axis=1)
        v_pad = jnp.pad(v, ((0, NPAD - n), (0, 0)))
        ab = _ab_tables(v_pad, w_ab, rb=NPR)
        c2 = _mlp2half(ea_pad, w1e_enc, b1e_enc, wc, cc, rb=1024)

        hs = _edge_sc(ab, c2, src3d, dst3d)

        v = _nodeupd(v, hs[0, :n], hs[1, :n], deg, w1va, wg, u,
                     b2d(p["psi_v"]["b1"]), p["psi_v"]["w2"],
                     b2d(p["psi_v"]["b2"]), rb=1000)

    pd = params["dec"]
    return _mlp(v, pd["w1"], b2d(pd["b1"]), pd["w2"], b2d(pd["b2"]), rb=1000)


# speedup vs baseline: 1.1173x; 1.1173x over previous
import functools

import jax
import jax.numpy as jnp
from jax import lax
from jax.experimental import pallas as pl
from jax.experimental.pallas import tpu as pltpu
from jax.experimental.pallas import tpu_sc as plsc

NC = 2
NS = 16
LN = 16
FH = 128
CH = 64
SB = 16
CPT = 320
EPT = CPT * CH
EPAD = NS * EPT
NPR = 632
NPAD = NS * NPR


def _mlp_body(x_ref, w1_ref, b1_ref, w2_ref, b2_ref, o_ref):
    h = jnp.maximum(
        jnp.dot(x_ref[...], w1_ref[...], preferred_element_type=jnp.float32)
        + b1_ref[...], 0.0)
    o_ref[...] = (
        jnp.dot(h, w2_ref[...], preferred_element_type=jnp.float32) + b2_ref[...])


def _mlp(x, w1, b1, w2, b2, rb):
    r, din = x.shape
    dh = w1.shape[1]
    dout = w2.shape[1]
    return pl.pallas_call(
        _mlp_body,
        grid=(r // rb,),
        in_specs=[
            pl.BlockSpec((rb, din), lambda i: (i, 0)),
            pl.BlockSpec((din, dh), lambda i: (0, 0)),
            pl.BlockSpec((1, dh), lambda i: (0, 0)),
            pl.BlockSpec((dh, dout), lambda i: (0, 0)),
            pl.BlockSpec((1, dout), lambda i: (0, 0)),
        ],
        out_specs=pl.BlockSpec((rb, dout), lambda i: (i, 0)),
        out_shape=jax.ShapeDtypeStruct((r, dout), jnp.float32),
    )(x, w1, b1, w2, b2)


def _mlp2half_body(x_ref, w1_ref, b1_ref, w2_ref, b2_ref, o_ref):
    h = jnp.maximum(
        jnp.dot(x_ref[...], w1_ref[...], preferred_element_type=jnp.float32)
        + b1_ref[...], 0.0)
    res = jnp.dot(h, w2_ref[...], preferred_element_type=jnp.float32) + b2_ref[...]
    o_ref[0] = res[:, :FH]
    o_ref[1] = res[:, FH:]


def _mlp2half(x, w1, b1, w2, b2, rb):
    r, din = x.shape
    dh = w1.shape[1]
    return pl.pallas_call(
        _mlp2half_body,
        grid=(r // rb,),
        in_specs=[
            pl.BlockSpec((rb, din), lambda i: (i, 0)),
            pl.BlockSpec((din, dh), lambda i: (0, 0)),
            pl.BlockSpec((1, dh), lambda i: (0, 0)),
            pl.BlockSpec((dh, 2 * FH), lambda i: (0, 0)),
            pl.BlockSpec((1, 2 * FH), lambda i: (0, 0)),
        ],
        out_specs=pl.BlockSpec((2, rb, FH), lambda i: (0, i, 0)),
        out_shape=jax.ShapeDtypeStruct((2, r, FH), jnp.float32),
    )(x, w1, b1, w2, b2)


def _ab_body(x_ref, w_ref, o_ref):
    ab = jnp.dot(x_ref[...], w_ref[...], preferred_element_type=jnp.float32)
    for k in range(4):
        o_ref[k] = ab[:, k * FH:(k + 1) * FH]


def _ab_tables(v, w_ab, rb):
    r = v.shape[0]
    return pl.pallas_call(
        _ab_body,
        grid=(r // rb,),
        in_specs=[
            pl.BlockSpec((rb, v.shape[1]), lambda i: (i, 0)),
            pl.BlockSpec((v.shape[1], 4 * FH), lambda i: (0, 0)),
        ],
        out_specs=pl.BlockSpec((4, rb, FH), lambda i: (0, i, 0)),
        out_shape=jax.ShapeDtypeStruct((4, r, FH), jnp.float32),
    )(v, w_ab)


def _wprep_body(w2e_ref, b2e_ref, w1e_ref, b1_ref, w2_ref, b2_ref, w1vb_ref,
                wc_ref, cc_ref, wg_ref, u_ref):
    wc_ref[...] = jnp.dot(w2e_ref[...], w1e_ref[...],
                          preferred_element_type=jnp.float32)
    cc_ref[...] = jnp.dot(b2e_ref[...], w1e_ref[...],
                          preferred_element_type=jnp.float32) + b1_ref[...]
    wg_ref[...] = jnp.dot(w2_ref[...], w1vb_ref[...],
                          preferred_element_type=jnp.float32)
    u_ref[...] = jnp.dot(b2_ref[...], w1vb_ref[...],
                         preferred_element_type=jnp.float32)


def _wprep(w2e, b2e, w1e, b1, w2, b2, w1vb):
    d = w2e.shape[0]
    return pl.pallas_call(
        _wprep_body,
        out_shape=[
            jax.ShapeDtypeStruct((d, d), jnp.float32),
            jax.ShapeDtypeStruct((1, d), jnp.float32),
            jax.ShapeDtypeStruct((d, d), jnp.float32),
            jax.ShapeDtypeStruct((1, d), jnp.float32),
        ],
    )(w2e, b2e, w1e, b1, w2, b2, w1vb)


def _nodeupd_body(v_ref, h0_ref, h1_ref, dg_ref, w1va_ref, wg_ref, u_ref,
                  b1v_ref, w2v_ref, b2v_ref, o_ref):
    wg = wg_ref[...]
    z = (jnp.dot(v_ref[...], w1va_ref[...], preferred_element_type=jnp.float32)
         + jnp.dot(h0_ref[...], wg[:FH, :], preferred_element_type=jnp.float32)
         + jnp.dot(h1_ref[...], wg[FH:, :], preferred_element_type=jnp.float32)
         + dg_ref[:, 0:1] * u_ref[...]
         + b1v_ref[...])
    o_ref[...] = (jnp.dot(jnp.maximum(z, 0.0), w2v_ref[...],
                          preferred_element_type=jnp.float32) + b2v_ref[...])


def _nodeupd(v, h0, h1, dg, w1va, wg, u, b1v, w2v, b2v, rb):
    r, d = v.shape
    return pl.pallas_call(
        _nodeupd_body,
        grid=(r // rb,),
        in_specs=[
            pl.BlockSpec((rb, d), lambda i: (i, 0)),
            pl.BlockSpec((rb, FH), lambda i: (i, 0)),
            pl.BlockSpec((rb, FH), lambda i: (i, 0)),
            pl.BlockSpec((rb, LN), lambda i: (i, 0)),
            pl.BlockSpec((d, d), lambda i: (0, 0)),
            pl.BlockSpec((d, d), lambda i: (0, 0)),
            pl.BlockSpec((1, d), lambda i: (0, 0)),
            pl.BlockSpec((1, d), lambda i: (0, 0)),
            pl.BlockSpec((d, d), lambda i: (0, 0)),
            pl.BlockSpec((1, d), lambda i: (0, 0)),
        ],
        out_specs=pl.BlockSpec((rb, d), lambda i: (i, 0)),
        out_shape=jax.ShapeDtypeStruct((r, d), jnp.float32),
    )(v, h0, h1, dg, w1va, wg, u, b1v, w2v, b2v)


def _edge_sc(ab, c2, src3d, dst3d):
    ng = CPT // SB
    mesh = plsc.VectorSubcoreMesh(core_axis_name="c", subcore_axis_name="s")

    def body(ab_hbm, c_hbm, src_hbm, dst_hbm, hs_out,
             idx_s, idx_d, a0, a1, b0, b1, cbuf, scat_idx,
             sa0, sa1, sb0, sb1, sw, acc):
        c = lax.axis_index("c")
        s = lax.axis_index("s")
        abuf = [a0, a1]
        bbuf = [b0, b1]
        sa = [sa0, sa1]
        sb = [sb0, sb1]

        def issue_gather(p, sidx, didx):
            pltpu.async_copy(ab_hbm.at[c].at[sidx], abuf[p], sa[p])
            pltpu.async_copy(ab_hbm.at[c + 2].at[didx], bbuf[p], sb[p])

        def wait_gather(p):
            pltpu.make_async_copy(ab_hbm.at[c].at[idx_s.at[0]],
                                  abuf[p], sa[p]).wait()
            pltpu.make_async_copy(ab_hbm.at[c].at[idx_d.at[0]],
                                  bbuf[p], sb[p]).wait()

        def wait_scatter(p):
            pltpu.make_async_copy(abuf[p], acc.at[scat_idx.at[p]], sw).wait()

        def copy_scat_idx(p, j2):
            for k in range(CH // LN):
                sl = pl.ds(k * LN, LN)
                scat_idx[p, sl] = idx_d[j2, sl]

        @plsc.parallel_loop(0, CH)
        def _fz(i):
            for k in range(FH // LN):
                cbuf[i, pl.ds(k * LN, LN)] = jnp.zeros((LN,), jnp.float32)

        def _zero(m, carry):
            pltpu.sync_copy(cbuf.at[pl.ds(0, 8)],
                            acc.at[pl.ds(s * NPR + m * 8, 8)])
            return carry
        lax.fori_loop(0, NPR // 8, _zero, 0)

        pltpu.sync_copy(src_hbm.at[s, pl.ds(0, SB)], idx_s)
        pltpu.sync_copy(dst_hbm.at[s, pl.ds(0, SB)], idx_d)
        copy_scat_idx(1, 0)
        plsc.subcore_barrier()
        pltpu.async_copy(cbuf, acc.at[scat_idx.at[1]], sw, add=True)
        issue_gather(0, idx_s.at[0], idx_d.at[0])

        def sb_body(g, carry):
            for j2 in range(SB):
                p = j2 & 1
                pltpu.sync_copy(
                    c_hbm.at[c, pl.ds(s * EPT + (g * SB + j2) * CH, CH)], cbuf)
                wait_gather(p)
                va = abuf[p]
                vb = bbuf[p]

                @plsc.parallel_loop(0, CH, unroll=2)
                def _row(r2):
                    for k in range(FH // LN):
                        sl = pl.ds(k * LN, LN)
                        va[r2, sl] = jnp.maximum(
                            va[r2, sl] + vb[r2, sl] + cbuf[r2, sl], 0.0)

                copy_scat_idx(p, j2)
                wait_scatter(1 - p)
                pltpu.async_copy(va, acc.at[scat_idx.at[p]], sw, add=True)
                if j2 < SB - 1:
                    issue_gather(1 - p, idx_s.at[j2 + 1], idx_d.at[j2 + 1])

            @pl.when(g < CPT // SB - 1)
            def _():
                pltpu.sync_copy(src_hbm.at[s, pl.ds((g + 1) * SB, SB)], idx_s)
                pltpu.sync_copy(dst_hbm.at[s, pl.ds((g + 1) * SB, SB)], idx_d)
                issue_gather(0, idx_s.at[0], idx_d.at[0])
            return carry
        lax.fori_loop(0, CPT // SB, sb_body, 0)

        wait_scatter(1)
        plsc.subcore_barrier()

        pltpu.sync_copy(acc.at[pl.ds(s * NPR, NPR)],
                        hs_out.at[c, pl.ds(s * NPR, NPR)])

    call = pl.kernel(
        body,
        out_type=jax.ShapeDtypeStruct((NC, NPAD, FH), jnp.float32),
        mesh=mesh,
        scratch_types=[
            pltpu.VMEM((SB, CH), jnp.int32),
            pltpu.VMEM((SB, CH), jnp.int32),
            pltpu.VMEM((CH, FH), jnp.float32),
            pltpu.VMEM((CH, FH), jnp.float32),
            pltpu.VMEM((CH, FH), jnp.float32),
            pltpu.VMEM((CH, FH), jnp.float32),
            pltpu.VMEM((CH, FH), jnp.float32),
            pltpu.VMEM((2, CH), jnp.int32),
            pltpu.SemaphoreType.DMA,
            pltpu.SemaphoreType.DMA,
            pltpu.SemaphoreType.DMA,
            pltpu.SemaphoreType.DMA,
            pltpu.SemaphoreType.DMA,
            pltpu.VMEM_SHARED((NPAD, FH), jnp.float32),
        ],
    )
    return call(ab, c2, src3d, dst3d)


def _deg_sc(dst3d):
    mesh = plsc.VectorSubcoreMesh(core_axis_name="c", subcore_axis_name="s")

    def body(dst_hbm, deg_out, idx_d, ones_b, zrow16, dacc):
        c = lax.axis_index("c")
        s = lax.axis_index("s")

        def _fill(i, carry):
            ones_b[i, :] = jnp.ones((LN,), jnp.float32)
            zrow16[i % 8, :] = jnp.zeros((LN,), jnp.float32)
            return carry
        lax.fori_loop(0, CH, _fill, 0)

        def _zero(m, carry):
            pltpu.sync_copy(zrow16, dacc.at[pl.ds(s * NPR + m * 8, 8)])
            return carry
        lax.fori_loop(0, NPR // 8, _zero, 0)
        plsc.subcore_barrier()

        def superblock(g, carry):
            pltpu.sync_copy(dst_hbm.at[s, pl.ds(g * SB, SB)], idx_d)

            def chunk(j2, carry2):
                pltpu.sync_copy(ones_b, dacc.at[idx_d.at[j2]], add=True)
                return carry2
            lax.fori_loop(0, SB, chunk, 0)
            return carry
        lax.fori_loop(0, CPT // SB, superblock, 0)
        plsc.subcore_barrier()

        @pl.when(c == 0)
        def _():
            pltpu.sync_copy(dacc.at[pl.ds(s * NPR, NPR)],
                            deg_out.at[pl.ds(s * NPR, NPR)])

    call = pl.kernel(
        body,
        out_type=jax.ShapeDtypeStruct((NPAD, LN), jnp.float32),
        mesh=mesh,
        scratch_types=[
            pltpu.VMEM((SB, CH), jnp.int32),
            pltpu.VMEM((CH, LN), jnp.float32),
            pltpu.VMEM((8, LN), jnp.float32),
            pltpu.VMEM_SHARED((NPAD, LN), jnp.float32),
        ],
    )
    return call(dst3d)


def kernel(vertexes, edge_index, edge_attr, params):
    n = vertexes.shape[0]
    e = edge_attr.shape[0]
    dh = params["enc_v"]["w2"].shape[1]

    def b2d(b):
        return b.reshape(1, -1)

    pad_e = EPAD - e
    idx_pad = jnp.full((2, pad_e), NPAD - 1, jnp.int32)
    eidx = jnp.concatenate([edge_index, idx_pad], axis=1)
    src3d = eidx[0].reshape(NS, CPT, CH)
    dst3d = eidx[1].reshape(NS, CPT, CH)
    ea_pad = jnp.pad(edge_attr, ((0, pad_e), (0, 0)))

    pv = params["enc_v"]
    v = _mlp(vertexes, pv["w1"], b2d(pv["b1"]), pv["w2"], b2d(pv["b2"]), rb=1000)

    pe = params["enc_e"]
    w1e_enc, b1e_enc = pe["w1"], b2d(pe["b1"])
    w2e_enc, b2e_enc = pe["w2"], b2d(pe["b2"])

    deg = _deg_sc(dst3d)[:n]

    for p in params["mpb"]:
        w1_pe = p["psi_e"]["w1"]
        w1s = w1_pe[:dh]
        w1d = w1_pe[dh:2 * dh]
        w1e = w1_pe[2 * dh:]
        w1_pv = p["psi_v"]["w1"]
        w1va = w1_pv[:dh]
        w1vb = w1_pv[dh:]

        wc, cc, wg, u = _wprep(
            w2e_enc, b2e_enc, w1e, b2d(p["psi_e"]["b1"]),
            p["psi_e"]["w2"], b2d(p["psi_e"]["b2"]), w1vb)

        w_ab = jnp.concatenate([w1s, w1d], axis=1)
        v_pad = jnp.pad(v, ((0, NPAD - n), (0, 0)))
        ab = _ab_tables(v_pad, w_ab, rb=NPR)
        c2 = _mlp2half(ea_pad, w1e_enc, b1e_enc, wc, cc, rb=1024)

        hs = _edge_sc(ab, c2, src3d, dst3d)

        v = _nodeupd(v, hs[0, :n], hs[1, :n], deg, w1va, wg, u,
                     b2d(p["psi_v"]["b1"]), p["psi_v"]["w2"],
                     b2d(p["psi_v"]["b2"]), rb=1000)

    pd = params["dec"]
    return _mlp(v, pd["w1"], b2d(pd["b1"]), pd["w2"], b2d(pd["b2"]), rb=1000)
